# baseline (speedup 1.0000x reference)
import functools

import jax
import jax.numpy as jnp
from jax import lax
from jax.experimental import pallas as pl
from jax.experimental.pallas import tpu as pltpu, tpu_sc as plsc

_SEQ = 2048
_D = 1024
_BATCH = 4

_info = plsc.get_sparse_core_info()
_NC, _NS = _info.num_cores, _info.num_subcores
_NW = _NC * _NS

_ROWS_PER_W = _SEQ // _NW
_CHUNK_ROWS = 16
_CHUNK = _CHUNK_ROWS * _D
_N_CHUNKS = _ROWS_PER_W // _CHUNK_ROWS
_N_VEC = _CHUNK // 16


def _body(x_hbm, pos_hbm, out_hbm, x_buf, pos_buf):
    wid = lax.axis_index("s") * _NC + lax.axis_index("c")
    base = wid * (_ROWS_PER_W * _D)
    for ch in range(_N_CHUNKS):
        off = base + ch * _CHUNK
        pltpu.sync_copy(pos_hbm.at[pl.ds(off, _CHUNK)], pos_buf)
        for b in range(_BATCH):
            pltpu.sync_copy(x_hbm.at[b, pl.ds(off, _CHUNK)], x_buf)

            def add_one(i, _):
                sl = pl.ds(i * 16, 16)
                x_buf[sl] = x_buf[sl] + pos_buf[sl]
                return 0

            lax.fori_loop(0, _N_VEC, add_one, 0)
            pltpu.sync_copy(x_buf, out_hbm.at[b, pl.ds(off, _CHUNK)])


_sc_add = functools.partial(
    pl.kernel,
    mesh=plsc.VectorSubcoreMesh(core_axis_name="c", subcore_axis_name="s"),
    out_type=jax.ShapeDtypeStruct((_BATCH, _SEQ * _D), jnp.float32),
    scratch_types=[
        pltpu.VMEM((_CHUNK,), jnp.float32),
        pltpu.VMEM((_CHUNK,), jnp.float32),
    ],
)(_body)


@jax.jit
def kernel(x, pos_table):
    b, t, d = x.shape
    xf = x.reshape(b, t * d)
    posf = pos_table.reshape(-1)
    out = _sc_add(xf, posf)
    return out.reshape(b, t, d)
